# baseline (speedup 1.0000x reference)
import functools

import jax
import jax.numpy as jnp
from jax import lax
from jax.experimental import pallas as pl
from jax.experimental.pallas import tpu as pltpu
from jax.experimental.pallas import tpu_sc as plsc

_N = 10000
_B = 640000
_NC = 2
_NS = 16
_NW = _NC * _NS
_BPW = _B // _NW
_L = 16


def _gather_body(z_hbm, rows_hbm, cols_hbm, out_hbm, r_v, c_v, idx_v, out_v, sem):
    wid = lax.axis_index("s") * _NC + lax.axis_index("c")
    base = wid * _BPW

    pltpu.sync_copy(rows_hbm.at[pl.ds(base, _BPW)], r_v)
    pltpu.sync_copy(cols_hbm.at[pl.ds(base, _BPW)], c_v)

    def flatten(i, carry):
        sl = pl.ds(i * _L, _L)
        idx_v[sl] = r_v[sl] * _N + c_v[sl]
        return carry

    lax.fori_loop(0, _BPW // _L, flatten, 0, unroll=8)

    pltpu.async_copy(z_hbm.at[idx_v], out_v, sem).wait()
    pltpu.sync_copy(out_v, out_hbm.at[pl.ds(base, _BPW)])


@jax.jit
def _reconstruct(zflat, rows, cols):
    mesh = plsc.VectorSubcoreMesh(core_axis_name="c", subcore_axis_name="s")
    return pl.kernel(
        _gather_body,
        mesh=mesh,
        out_type=jax.ShapeDtypeStruct((_B,), jnp.float32),
        scratch_types=[
            pltpu.VMEM((_BPW,), jnp.int32),
            pltpu.VMEM((_BPW,), jnp.int32),
            pltpu.VMEM((_BPW,), jnp.int32),
            pltpu.VMEM((_BPW,), jnp.float32),
            pltpu.SemaphoreType.DMA,
        ],
    )(zflat, rows, cols)


def kernel(z, edge_index):
    zflat = z.reshape(-1)
    return _reconstruct(zflat, edge_index[0], edge_index[1])
